# Initial kernel scaffold; baseline (speedup 1.0000x reference)
import jax
import jax.numpy as jnp
from jax.experimental import pallas as pl


def kernel(seqs, exps, W):
    raise NotImplementedError("write your pallas kernel here")


# speedup vs baseline: 3.9134x; 3.9134x over previous
import functools

import jax
import jax.numpy as jnp
from jax import lax
from jax.experimental import pallas as pl
from jax.experimental.pallas import tpu as pltpu
from jax.experimental.pallas import tpu_sc as plsc

VOCAB = 457
EMB = 64
N = 1024 * 457
CH = 128
NCH = N // CH

_info = plsc.get_sparse_core_info()
_NC = _info.num_cores
_NS = _info.num_subcores
NW = _NC * _NS

_mesh = plsc.VectorSubcoreMesh(core_axis_name="c", subcore_axis_name="s")


@functools.partial(
    pl.kernel,
    mesh=_mesh,
    out_type=(
        jax.ShapeDtypeStruct((N, EMB), jnp.float32),
        jax.ShapeDtypeStruct((N, EMB), jnp.float32),
    ),
    scratch_types=[
        pltpu.VMEM((CH,), jnp.int32),
        pltpu.VMEM((CH, EMB), jnp.float32),
        pltpu.SemaphoreType.DMA,
    ],
    compiler_params=pltpu.CompilerParams(use_tc_tiling_on_sc=False),
)
def _lookup(seq_idx, exp_idx, table, seq_out, exp_out, idx_v, rows_v, sem):
    wid = lax.axis_index("s") * _NC + lax.axis_index("c")

    def run(idx_hbm, out_hbm):
        n_chunks = (NCH - wid + NW - 1) // NW

        def body(i, carry):
            off = (wid + i * NW) * CH
            pltpu.sync_copy(idx_hbm.at[pl.ds(off, CH)], idx_v)
            pltpu.async_copy(table.at[idx_v], rows_v, sem).wait()
            pltpu.sync_copy(rows_v, out_hbm.at[pl.ds(off, CH)])
            return carry

        lax.fori_loop(0, n_chunks, body, 0)

    run(seq_idx, seq_out)
    run(exp_idx, exp_out)


def kernel(seqs, exps, W):
    b, s = seqs.shape
    seq_o, exp_o = _lookup(seqs.reshape(-1), exps.reshape(-1), W)
    return (seq_o.reshape(b, s, EMB), exp_o.reshape(b, s, EMB))


# speedup vs baseline: 4.2206x; 1.0785x over previous
import functools

import jax
import jax.numpy as jnp
from jax import lax
from jax.experimental import pallas as pl
from jax.experimental.pallas import tpu as pltpu
from jax.experimental.pallas import tpu_sc as plsc

VOCAB = 457
EMB = 64
N = 1024 * 457

_info = plsc.get_sparse_core_info()
_NC = _info.num_cores
_NS = _info.num_subcores
NW = _NC * _NS

PW = N // NW
CH = 512
FULL = PW // CH
TAIL = PW - FULL * CH

_mesh = plsc.VectorSubcoreMesh(core_axis_name="c", subcore_axis_name="s")


@functools.partial(
    pl.kernel,
    mesh=_mesh,
    out_type=(
        jax.ShapeDtypeStruct((N, EMB), jnp.float32),
        jax.ShapeDtypeStruct((N, EMB), jnp.float32),
    ),
    scratch_types=[
        pltpu.VMEM((PW,), jnp.int32),
        pltpu.VMEM((CH, EMB), jnp.float32),
        pltpu.VMEM((CH, EMB), jnp.float32),
        pltpu.SemaphoreType.DMA,
        pltpu.SemaphoreType.DMA,
        pltpu.SemaphoreType.DMA,
        pltpu.SemaphoreType.DMA,
    ],
    compiler_params=pltpu.CompilerParams(use_tc_tiling_on_sc=False),
)
def _lookup(seq_idx, exp_idx, table, seq_out, exp_out,
            idxv, r0, r1, gs0, gs1, ss0, ss1):
    wid = lax.axis_index("s") * _NC + lax.axis_index("c")
    base = wid * PW

    rbuf = (r0, r1)
    gsem = (gs0, gs1)
    ssem = (ss0, ss1)

    def run(idx_hbm, out_hbm):
        pltpu.sync_copy(idx_hbm.at[pl.ds(base, PW)], idxv)

        def start_g(j, b):
            pltpu.async_copy(
                table.at[idxv.at[pl.ds(j * CH, CH)]], rbuf[b], gsem[b])

        def start_s(j, b):
            pltpu.async_copy(
                rbuf[b], out_hbm.at[pl.ds(base + j * CH, CH)], ssem[b])

        def wait_g(b):
            pltpu.make_async_copy(
                table.at[idxv.at[pl.ds(0, CH)]], rbuf[b], gsem[b]).wait()

        def wait_s(b):
            pltpu.make_async_copy(
                rbuf[b], out_hbm.at[pl.ds(base, CH)], ssem[b]).wait()

        start_g(0, 0)
        start_g(1, 1)
        wait_g(0)
        start_s(0, 0)

        def body(k, carry):
            j = 2 * k
            wait_s(0)
            start_g(j, 0)
            wait_g(1)
            start_s(j - 1, 1)
            wait_s(1)
            start_g(j + 1, 1)
            wait_g(0)
            start_s(j, 0)
            return carry

        lax.fori_loop(1, FULL // 2, body, 0)

        wait_g(1)
        start_s(FULL - 1, 1)

        wait_s(0)
        pltpu.async_copy(
            table.at[idxv.at[pl.ds(FULL * CH, TAIL)]],
            r0.at[pl.ds(0, TAIL)], gs0).wait()
        tail_s = pltpu.async_copy(
            r0.at[pl.ds(0, TAIL)],
            out_hbm.at[pl.ds(base + FULL * CH, TAIL)], ss0)
        wait_s(1)
        tail_s.wait()

    run(seq_idx, seq_out)
    run(exp_idx, exp_out)


def kernel(seqs, exps, W):
    b, s = seqs.shape
    seq_o, exp_o = _lookup(seqs.reshape(-1), exps.reshape(-1), W)
    return (seq_o.reshape(b, s, EMB), exp_o.reshape(b, s, EMB))
